# Initial kernel scaffold; baseline (speedup 1.0000x reference)
import jax
import jax.numpy as jnp
from jax.experimental import pallas as pl

_K = 1024
_BIG = jnp.int32(_K)


def _vq_body(x_ref, cb_ref, idx_ref, qt_ref):
    x = x_ref[0]
    cb = cb_ref[...]
    hw = x.shape[1]
    ab = jax.lax.dot_general(cb, x, (((1,), (0,)), ((), ())),
                             preferred_element_type=jnp.float32)
    b_sq = jnp.sum(cb * cb, axis=1, keepdims=True)
    score = b_sq - 2.0 * ab
    mins = jnp.min(score, axis=0, keepdims=True)
    kio = jax.lax.broadcasted_iota(jnp.int32, (_K, hw), 0)
    idx = jnp.min(jnp.where(score == mins, kio, _BIG), axis=0)
    idx_ref[0, 0, :] = idx
    onehot = (kio == idx[None, :]).astype(jnp.float32)
    qt = jax.lax.dot_general(cb, onehot, (((0,), (0,)), ((), ())),
                             preferred_element_type=jnp.float32)
    qt_ref[0] = qt


def kernel(laten, codebook):
    b_s, c, h, w = laten.shape
    hw = h * w
    x = laten.reshape(b_s, c, hw)
    idx3, qt = pl.pallas_call(
        _vq_body,
        grid=(b_s,),
        in_specs=[
            pl.BlockSpec((1, c, hw), lambda b: (b, 0, 0)),
            pl.BlockSpec((_K, c), lambda b: (0, 0)),
        ],
        out_specs=[
            pl.BlockSpec((1, 1, hw), lambda b: (b, 0, 0)),
            pl.BlockSpec((1, c, hw), lambda b: (b, 0, 0)),
        ],
        out_shape=[
            jax.ShapeDtypeStruct((b_s, 1, hw), jnp.int32),
            jax.ShapeDtypeStruct((b_s, c, hw), jnp.float32),
        ],
    )(x, codebook)
    return idx3.reshape(b_s, h, w), qt.reshape(b_s, c, h, w)


# speedup vs baseline: 2.1481x; 2.1481x over previous
import jax
import jax.numpy as jnp
from jax.experimental import pallas as pl

_K = 1024


def _vq_body(x_ref, cb_ref, idx_ref, qt_ref):
    x = x_ref[0]
    cb = cb_ref[...]
    hw = x.shape[1]
    ab = jax.lax.dot_general(cb, x, (((1,), (0,)), ((), ())),
                             preferred_element_type=jnp.float32)
    b_sq = jnp.sum(cb * cb, axis=1, keepdims=True)
    score = b_sq - 2.0 * ab
    mins = jnp.min(score, axis=0, keepdims=True)
    kio = jax.lax.broadcasted_iota(jnp.int32, (_K, hw), 0)
    idx = jnp.min(jnp.where(score == mins, kio, _K), axis=0)
    idx_ref[0, 0, :] = idx
    onehot = (kio == idx[None, :]).astype(jnp.float32)
    qt = jax.lax.dot_general(cb, onehot, (((0,), (0,)), ((), ())),
                             preferred_element_type=jnp.float32)
    qt_ref[0] = qt


def kernel(laten, codebook):
    b_s, c, h, w = laten.shape
    hw = h * w
    x = laten.reshape(b_s, c, hw)
    idx3, qt = pl.pallas_call(
        _vq_body,
        grid=(b_s,),
        in_specs=[
            pl.BlockSpec((1, c, hw), lambda b: (b, 0, 0)),
            pl.BlockSpec((_K, c), lambda b: (0, 0)),
        ],
        out_specs=[
            pl.BlockSpec((1, 1, hw), lambda b: (b, 0, 0)),
            pl.BlockSpec((1, c, hw), lambda b: (b, 0, 0)),
        ],
        out_shape=[
            jax.ShapeDtypeStruct((b_s, 1, hw), jnp.int32),
            jax.ShapeDtypeStruct((b_s, c, hw), jnp.float32),
        ],
    )(x, codebook)
    return idx3.reshape(b_s, h, w), qt.reshape(b_s, c, h, w)


# speedup vs baseline: 2.5500x; 1.1871x over previous
import jax
import jax.numpy as jnp
from jax.experimental import pallas as pl

_K = 1024


def _vq_body(x_ref, cb1_ref, cb2_ref, idx_ref, qt_ref):
    x = x_ref[0]
    hw = x.shape[1]
    c = x.shape[0]
    ones_row = jnp.full((1, hw), 1.0, dtype=jnp.float32)
    x_aug = jnp.concatenate([x, ones_row], axis=0)
    score = jax.lax.dot_general(cb1_ref[...], x_aug, (((1,), (0,)), ((), ())),
                                preferred_element_type=jnp.float32)
    mins = jnp.min(score, axis=0, keepdims=True)
    onehot = jnp.where(score == mins, 1.0, 0.0)
    qa = jax.lax.dot_general(cb2_ref[...], onehot, (((0,), (0,)), ((), ())),
                             preferred_element_type=jnp.float32)
    idx_ref[0, 0, :] = qa[c, :].astype(jnp.int32)
    qt_ref[0] = qa[:c, :]


def kernel(laten, codebook):
    b_s, c, h, w = laten.shape
    hw = h * w
    x = laten.reshape(b_s, c, hw)
    b_sq = jnp.sum(codebook * codebook, axis=1, keepdims=True)
    kio = jax.lax.iota(jnp.float32, _K)[:, None]
    cb1 = jnp.concatenate([-2.0 * codebook, b_sq], axis=1)
    cb2 = jnp.concatenate([codebook, kio], axis=1)
    idx3, qt = pl.pallas_call(
        _vq_body,
        grid=(b_s,),
        in_specs=[
            pl.BlockSpec((1, c, hw), lambda b: (b, 0, 0)),
            pl.BlockSpec((_K, c + 1), lambda b: (0, 0)),
            pl.BlockSpec((_K, c + 1), lambda b: (0, 0)),
        ],
        out_specs=[
            pl.BlockSpec((1, 1, hw), lambda b: (b, 0, 0)),
            pl.BlockSpec((1, c, hw), lambda b: (b, 0, 0)),
        ],
        out_shape=[
            jax.ShapeDtypeStruct((b_s, 1, hw), jnp.int32),
            jax.ShapeDtypeStruct((b_s, c, hw), jnp.float32),
        ],
    )(x, cb1, cb2)
    return idx3.reshape(b_s, h, w), qt.reshape(b_s, c, h, w)
